# baseline (speedup 1.0000x reference)
import functools

import jax
import jax.numpy as jnp
from jax import lax
from jax.experimental import pallas as pl
from jax.experimental.pallas import tpu as pltpu
from jax.experimental.pallas import tpu_sc as plsc

N_NODES = 10000
N_EDGES = 320000
D = 128
G = 100
OUT = 16
L = 4
BN_EPS = 1e-5

NC = 2
NS = 16
NW = NC * NS
CHUNK = 80
CPT = 126
GRP = 21
NGRP = CPT // GRP
E_PAD = NW * CPT * CHUNK
ACC_ROWS = N_NODES + 8
RCH = 80
NRCH = N_NODES // RCH


def _sc_body(h_hbm, src_hbm, dst_hbm, out_hbm,
             sidxA, didxA, sidxB, didxB, rowsA, rowsB, rowsC,
             semA, semB, semC, semI, acc):
    c = lax.axis_index("c")
    s = lax.axis_index("s")
    w = c * NS + s

    def _zrow(r, _):
        for cc in range(D // 16):
            rowsA[r, pl.ds(cc * 16, 16)] = jnp.zeros((16,), jnp.float32)
        return _
    lax.fori_loop(0, RCH, _zrow, None)
    for k in range((NRCH + NS - 1) // NS):
        j = s + NS * k

        @pl.when(j < NRCH)
        def _():
            pltpu.sync_copy(rowsA, acc.at[pl.ds(j * RCH, RCH)])

    plsc.subcore_barrier()

    def _gather(idx_ref, j, buf, sem):
        pltpu.async_copy(h_hbm.at[idx_ref.at[j]], buf, sem)

    def _wait(buf, sem):
        pltpu.make_async_copy(h_hbm.at[pl.ds(0, CHUNK)], buf, sem).wait()

    def _scat(idx_ref, j, buf):
        pltpu.sync_copy(buf, acc.at[idx_ref.at[j]], add=True)

    idx_pairs = [(sidxA, didxA), (sidxB, didxB)]
    bufs = [(rowsA, semA), (rowsB, semB), (rowsC, semC)]

    pltpu.sync_copy(src_hbm.at[w, 0], sidxA)
    pltpu.sync_copy(dst_hbm.at[w, 0], didxA)
    for g in range(NGRP):
        si, di = idx_pairs[g % 2]
        sn, dn = idx_pairs[(g + 1) % 2]
        if g + 1 < NGRP:
            pltpu.async_copy(src_hbm.at[w, g + 1], sn, semI)
            pltpu.async_copy(dst_hbm.at[w, g + 1], dn, semI)

        _gather(si, 0, rowsA, semA)

        def _pair(k, _):
            _gather(si, 2 * k + 1, rowsB, semB)
            _wait(rowsA, semA)
            _scat(di, 2 * k, rowsA)
            _gather(si, 2 * k + 2, rowsA, semA)
            _wait(rowsB, semB)
            _scat(di, 2 * k + 1, rowsB)
            return _
        lax.fori_loop(0, (GRP - 1) // 2, _pair, None)
        _wait(rowsA, semA)
        _scat(di, GRP - 1, rowsA)

        if g + 1 < NGRP:
            pltpu.make_async_copy(src_hbm.at[w, 0], sn, semI).wait()
            pltpu.make_async_copy(dst_hbm.at[w, 0], dn, semI).wait()

    plsc.subcore_barrier()

    for k in range((NRCH + NS - 1) // NS):
        j = s + NS * k

        @pl.when(j < NRCH)
        def _():
            pltpu.sync_copy(acc.at[pl.ds(j * RCH, RCH)], rowsA)
            pltpu.sync_copy(rowsA, out_hbm.at[c, pl.ds(j * RCH, RCH)])


_sc_scatter = functools.partial(
    pl.kernel,
    mesh=plsc.VectorSubcoreMesh(
        core_axis_name="c", subcore_axis_name="s", num_cores=NC, num_subcores=NS
    ),
    out_type=jax.ShapeDtypeStruct((NC, N_NODES, D), jnp.float32),
    scratch_types=[
        pltpu.VMEM((GRP, CHUNK), jnp.int32),
        pltpu.VMEM((GRP, CHUNK), jnp.int32),
        pltpu.VMEM((GRP, CHUNK), jnp.int32),
        pltpu.VMEM((GRP, CHUNK), jnp.int32),
        pltpu.VMEM((CHUNK, D), jnp.float32),
        pltpu.VMEM((CHUNK, D), jnp.float32),
        pltpu.VMEM((CHUNK, D), jnp.float32),
        pltpu.SemaphoreType.DMA,
        pltpu.SemaphoreType.DMA,
        pltpu.SemaphoreType.DMA,
        pltpu.SemaphoreType.DMA,
        pltpu.VMEM_SHARED((ACC_ROWS, D), jnp.float32),
    ],
)(_sc_body)


RB = 1000
NB = N_NODES // RB

_HI = jax.lax.Precision.DEFAULT


def _tc_layer_body(p_ref, h_ref, z_ref, W1_ref, b1_ref, W2_ref, b2_ref,
                   pWa_ref, pWb_ref, hout_ref, zout_ref):
    h = h_ref[...]
    pooled = p_ref[0] + p_ref[1] + h
    t = jnp.dot(pooled, W1_ref[...], preferred_element_type=jnp.float32,
                precision=_HI) + b1_ref[...]
    t = jnp.maximum(t, 0.0)
    hn = jnp.dot(t, W2_ref[...], preferred_element_type=jnp.float32,
                 precision=_HI) + b2_ref[...]
    hn = jnp.maximum(hn, 0.0)
    z = z_ref[...]
    z = z + jnp.dot(h, pWb_ref[...], preferred_element_type=jnp.float32,
                    precision=_HI)
    z = z + jnp.dot(hn, pWa_ref[...], preferred_element_type=jnp.float32,
                    precision=_HI)
    hout_ref[...] = hn
    zout_ref[...] = z


def _tc_layer(p, h, z, W1f, b1f, W2f, b2f, pWa, pWb):
    full = lambda shp: pl.BlockSpec(shp, lambda i: (0,) * len(shp))
    return pl.pallas_call(
        _tc_layer_body,
        grid=(NB,),
        in_specs=[
            pl.BlockSpec((NC, RB, D), lambda i: (0, i, 0)),
            pl.BlockSpec((RB, D), lambda i: (i, 0)),
            pl.BlockSpec((RB, OUT), lambda i: (i, 0)),
            full((D, D)), full((1, D)), full((D, D)), full((1, D)),
            full((D, OUT)), full((D, OUT)),
        ],
        out_specs=[
            pl.BlockSpec((RB, D), lambda i: (i, 0)),
            pl.BlockSpec((RB, OUT), lambda i: (i, 0)),
        ],
        out_shape=[
            jax.ShapeDtypeStruct((N_NODES, D), jnp.float32),
            jax.ShapeDtypeStruct((N_NODES, OUT), jnp.float32),
        ],
    )(p, h, z, W1f, b1f, W2f, b2f, pWa, pWb)


def _tc_seg_body(z_ref, b_ref, pb_ref, out_ref):
    i = pl.program_id(0)

    @pl.when(i == 0)
    def _():
        bias = jnp.sum(pb_ref[...], axis=0)
        out_ref[...] = jnp.broadcast_to(bias[None, :], (G, OUT))

    bvec = b_ref[0, 0, :]
    onehot = (bvec[:, None] == lax.broadcasted_iota(jnp.int32, (RB, G), 1))
    seg = lax.dot_general(onehot.astype(jnp.float32), z_ref[...],
                          (((0,), (0,)), ((), ())),
                          preferred_element_type=jnp.float32, precision=_HI)
    out_ref[...] += seg


def _tc_segment(z, batch3d, pred_b):
    return pl.pallas_call(
        _tc_seg_body,
        grid=(NB,),
        in_specs=[
            pl.BlockSpec((RB, OUT), lambda i: (i, 0)),
            pl.BlockSpec((1, 1, RB), lambda i: (i, 0, 0)),
            pl.BlockSpec((L + 1, OUT), lambda i: (0, 0)),
        ],
        out_specs=pl.BlockSpec((G, OUT), lambda i: (0, 0)),
        out_shape=jax.ShapeDtypeStruct((G, OUT), jnp.float32),
    )(z, batch3d, pred_b)


def kernel(x, edge_index, batch, mlp_W1, mlp_b1, bn1_g, bn1_b, mlp_W2, mlp_b2,
           gbn_g, gbn_b, pred_W, pred_b):
    pad = E_PAD - N_EDGES
    src2d = jnp.concatenate(
        [edge_index[1].astype(jnp.int32), jnp.zeros((pad,), jnp.int32)]
    ).reshape(NW, NGRP, GRP, CHUNK)
    dst2d = jnp.concatenate(
        [edge_index[0].astype(jnp.int32), jnp.full((pad,), N_NODES, jnp.int32)]
    ).reshape(NW, NGRP, GRP, CHUNK)
    batch3d = batch.astype(jnp.int32).reshape(NB, 1, RB)

    s = 1.0 / jnp.sqrt(1.0 + BN_EPS)
    W1f = mlp_W1 * (bn1_g * s)[:, None, :]
    b1f = (mlp_b1 * bn1_g * s + bn1_b)[:, None, :]
    W2f = mlp_W2 * (gbn_g * s)[:, None, :]
    b2f = (mlp_b2 * gbn_g * s + gbn_b)[:, None, :]
    pW0_zero = jnp.zeros((D, OUT), jnp.float32)

    h = x
    z = jnp.zeros((N_NODES, OUT), jnp.float32)
    for l in range(L):
        p = _sc_scatter(h, src2d, dst2d)
        pWb = pred_W[0] if l == 0 else pW0_zero
        h, z = _tc_layer(p, h, z, W1f[l], b1f[l], W2f[l], b2f[l],
                         pred_W[l + 1], pWb)
    return _tc_segment(z, batch3d, pred_b)


# speedup vs baseline: 1.7891x; 1.7891x over previous
import functools

import jax
import jax.numpy as jnp
from jax import lax
from jax.experimental import pallas as pl
from jax.experimental.pallas import tpu as pltpu
from jax.experimental.pallas import tpu_sc as plsc

N_NODES = 10000
N_EDGES = 320000
D = 128
G = 100
OUT = 16
L = 4
BN_EPS = 1e-5

NC = 2
NS = 16
NW = NC * NS
CHUNK = 80
CPT = N_EDGES // NW // CHUNK
GRP = 25
NGRP = CPT // GRP
ACC_ROWS = N_NODES
RCH = 80
NRCH = N_NODES // RCH


def _sc_body(h_hbm, src_hbm, dst_hbm, out_hbm,
             sidxA, didxA, sidxB, didxB, rowsA, rowsB, semA, semB, semI, acc):
    c = lax.axis_index("c")
    s = lax.axis_index("s")
    w = c * NS + s

    def _zrow(r, _):
        for cc in range(D // 16):
            rowsA[r, pl.ds(cc * 16, 16)] = jnp.zeros((16,), jnp.float32)
        return _
    lax.fori_loop(0, RCH, _zrow, None)
    for k in range((NRCH + NS - 1) // NS):
        j = s + NS * k

        @pl.when(j < NRCH)
        def _():
            pltpu.sync_copy(rowsA, acc.at[pl.ds(j * RCH, RCH)])

    plsc.subcore_barrier()

    def _gather(idx_ref, j, buf, sem):
        pltpu.async_copy(h_hbm.at[idx_ref.at[j]], buf, sem)

    def _wait(buf, sem):
        pltpu.make_async_copy(h_hbm.at[pl.ds(0, CHUNK)], buf, sem).wait()

    def _scat(idx_ref, j, buf):
        pltpu.sync_copy(buf, acc.at[idx_ref.at[j]], add=True)

    idx_pairs = [(sidxA, didxA), (sidxB, didxB)]

    pltpu.sync_copy(src_hbm.at[w, 0], sidxA)
    pltpu.sync_copy(dst_hbm.at[w, 0], didxA)
    for g in range(NGRP):
        si, di = idx_pairs[g % 2]
        sn, dn = idx_pairs[(g + 1) % 2]
        if g + 1 < NGRP:
            pltpu.async_copy(src_hbm.at[w, g + 1], sn, semI)
            pltpu.async_copy(dst_hbm.at[w, g + 1], dn, semI)

        _gather(si, 0, rowsA, semA)

        def _pair(k, _):
            _gather(si, 2 * k + 1, rowsB, semB)
            _wait(rowsA, semA)
            _scat(di, 2 * k, rowsA)
            _gather(si, 2 * k + 2, rowsA, semA)
            _wait(rowsB, semB)
            _scat(di, 2 * k + 1, rowsB)
            return _
        lax.fori_loop(0, (GRP - 1) // 2, _pair, None)
        _wait(rowsA, semA)
        _scat(di, GRP - 1, rowsA)

        if g + 1 < NGRP:
            pltpu.make_async_copy(src_hbm.at[w, 0], sn, semI).wait()
            pltpu.make_async_copy(dst_hbm.at[w, 0], dn, semI).wait()

    plsc.subcore_barrier()

    for k in range((NRCH + NS - 1) // NS):
        j = s + NS * k

        @pl.when(j < NRCH)
        def _():
            pltpu.sync_copy(acc.at[pl.ds(j * RCH, RCH)], rowsA)
            pltpu.sync_copy(rowsA, out_hbm.at[c, pl.ds(j * RCH, RCH)])


_sc_scatter = functools.partial(
    pl.kernel,
    mesh=plsc.VectorSubcoreMesh(
        core_axis_name="c", subcore_axis_name="s", num_cores=NC, num_subcores=NS
    ),
    out_type=jax.ShapeDtypeStruct((NC, N_NODES, D), jnp.float32),
    scratch_types=[
        pltpu.VMEM((GRP, CHUNK), jnp.int32),
        pltpu.VMEM((GRP, CHUNK), jnp.int32),
        pltpu.VMEM((GRP, CHUNK), jnp.int32),
        pltpu.VMEM((GRP, CHUNK), jnp.int32),
        pltpu.VMEM((CHUNK, D), jnp.float32),
        pltpu.VMEM((CHUNK, D), jnp.float32),
        pltpu.SemaphoreType.DMA,
        pltpu.SemaphoreType.DMA,
        pltpu.SemaphoreType.DMA,
        pltpu.VMEM_SHARED((ACC_ROWS, D), jnp.float32),
    ],
)(_sc_body)


RB = 1000
NB = N_NODES // RB

_HI = jax.lax.Precision.DEFAULT


def _tc_layer_body(p_ref, h_ref, z_ref, W1_ref, b1_ref, W2_ref, b2_ref,
                   pWa_ref, pWb_ref, hout_ref, zout_ref):
    h = h_ref[...]
    pooled = p_ref[0] + p_ref[1] + h
    t = jnp.dot(pooled, W1_ref[...], preferred_element_type=jnp.float32,
                precision=_HI) + b1_ref[...]
    t = jnp.maximum(t, 0.0)
    hn = jnp.dot(t, W2_ref[...], preferred_element_type=jnp.float32,
                 precision=_HI) + b2_ref[...]
    hn = jnp.maximum(hn, 0.0)
    z = z_ref[...]
    z = z + jnp.dot(h, pWb_ref[...], preferred_element_type=jnp.float32,
                    precision=_HI)
    z = z + jnp.dot(hn, pWa_ref[...], preferred_element_type=jnp.float32,
                    precision=_HI)
    hout_ref[...] = hn
    zout_ref[...] = z


def _tc_layer(p, h, z, W1f, b1f, W2f, b2f, pWa, pWb):
    full = lambda shp: pl.BlockSpec(shp, lambda i: (0,) * len(shp))
    return pl.pallas_call(
        _tc_layer_body,
        grid=(NB,),
        in_specs=[
            pl.BlockSpec((NC, RB, D), lambda i: (0, i, 0)),
            pl.BlockSpec((RB, D), lambda i: (i, 0)),
            pl.BlockSpec((RB, OUT), lambda i: (i, 0)),
            full((D, D)), full((1, D)), full((D, D)), full((1, D)),
            full((D, OUT)), full((D, OUT)),
        ],
        out_specs=[
            pl.BlockSpec((RB, D), lambda i: (i, 0)),
            pl.BlockSpec((RB, OUT), lambda i: (i, 0)),
        ],
        out_shape=[
            jax.ShapeDtypeStruct((N_NODES, D), jnp.float32),
            jax.ShapeDtypeStruct((N_NODES, OUT), jnp.float32),
        ],
    )(p, h, z, W1f, b1f, W2f, b2f, pWa, pWb)


def _tc_last_body(p_ref, h_ref, z_ref, W1_ref, b1_ref, W2_ref, b2_ref,
                  pWa_ref, b_ref, pb_ref, out_ref):
    i = pl.program_id(0)

    @pl.when(i == 0)
    def _():
        bias = jnp.sum(pb_ref[...], axis=0)
        out_ref[...] = jnp.broadcast_to(bias[None, :], (G, OUT))

    h = h_ref[...]
    pooled = p_ref[0] + p_ref[1] + h
    t = jnp.dot(pooled, W1_ref[...], preferred_element_type=jnp.float32,
                precision=_HI) + b1_ref[...]
    t = jnp.maximum(t, 0.0)
    hn = jnp.dot(t, W2_ref[...], preferred_element_type=jnp.float32,
                 precision=_HI) + b2_ref[...]
    hn = jnp.maximum(hn, 0.0)
    z = z_ref[...] + jnp.dot(hn, pWa_ref[...],
                             preferred_element_type=jnp.float32, precision=_HI)
    bvec = b_ref[0, 0, :]
    onehot = (bvec[:, None] == lax.broadcasted_iota(jnp.int32, (RB, G), 1))
    seg = lax.dot_general(onehot.astype(jnp.float32), z,
                          (((0,), (0,)), ((), ())),
                          preferred_element_type=jnp.float32, precision=_HI)
    out_ref[...] += seg


def _tc_last(p, h, z, W1f, b1f, W2f, b2f, pWa, batch3d, pred_b):
    full = lambda shp: pl.BlockSpec(shp, lambda i: (0,) * len(shp))
    return pl.pallas_call(
        _tc_last_body,
        grid=(NB,),
        in_specs=[
            pl.BlockSpec((NC, RB, D), lambda i: (0, i, 0)),
            pl.BlockSpec((RB, D), lambda i: (i, 0)),
            pl.BlockSpec((RB, OUT), lambda i: (i, 0)),
            full((D, D)), full((1, D)), full((D, D)), full((1, D)),
            full((D, OUT)),
            pl.BlockSpec((1, 1, RB), lambda i: (i, 0, 0)),
            full((L + 1, OUT)),
        ],
        out_specs=pl.BlockSpec((G, OUT), lambda i: (0, 0)),
        out_shape=jax.ShapeDtypeStruct((G, OUT), jnp.float32),
    )(p, h, z, W1f, b1f, W2f, b2f, pWa, batch3d, pred_b)


def kernel(x, edge_index, batch, mlp_W1, mlp_b1, bn1_g, bn1_b, mlp_W2, mlp_b2,
           gbn_g, gbn_b, pred_W, pred_b):
    src2d = edge_index[1].astype(jnp.int32).reshape(NW, NGRP, GRP, CHUNK)
    dst2d = edge_index[0].astype(jnp.int32).reshape(NW, NGRP, GRP, CHUNK)
    batch3d = batch.astype(jnp.int32).reshape(NB, 1, RB)

    s = 1.0 / jnp.sqrt(1.0 + BN_EPS)
    W1f = mlp_W1 * (bn1_g * s)[:, None, :]
    b1f = (mlp_b1 * bn1_g * s + bn1_b)[:, None, :]
    W2f = mlp_W2 * (gbn_g * s)[:, None, :]
    b2f = (mlp_b2 * gbn_g * s + gbn_b)[:, None, :]
    pW0_zero = jnp.zeros((D, OUT), jnp.float32)

    h = x
    z = jnp.zeros((N_NODES, OUT), jnp.float32)
    for l in range(L - 1):
        p = _sc_scatter(h, src2d, dst2d)
        pWb = pred_W[0] if l == 0 else pW0_zero
        h, z = _tc_layer(p, h, z, W1f[l], b1f[l], W2f[l], b2f[l],
                         pred_W[l + 1], pWb)
    p = _sc_scatter(h, src2d, dst2d)
    return _tc_last(p, h, z, W1f[L - 1], b1f[L - 1], W2f[L - 1], b2f[L - 1],
                    pred_W[L], batch3d, pred_b)


# speedup vs baseline: 1.8256x; 1.0204x over previous
import functools

import jax
import jax.numpy as jnp
from jax import lax
from jax.experimental import pallas as pl
from jax.experimental.pallas import tpu as pltpu
from jax.experimental.pallas import tpu_sc as plsc

N_NODES = 10000
N_EDGES = 320000
D = 128
G = 100
OUT = 16
L = 4
BN_EPS = 1e-5

NC = 2
NS = 16
NW = NC * NS
CHUNK = 80
CPT = N_EDGES // NW // CHUNK
GRP = 25
NGRP = CPT // GRP
ACC_ROWS = N_NODES
RCH = 80
NRCH = N_NODES // RCH


def _sc_body(h_hbm, src_hbm, dst_hbm, out_hbm,
             sidxA, didxA, sidxB, didxB, rowsA, rowsB, semA, semB, semI, acc):
    c = lax.axis_index("c")
    s = lax.axis_index("s")
    w = c * NS + s

    def _zrow(r, _):
        for cc in range(D // 16):
            rowsA[r, pl.ds(cc * 16, 16)] = jnp.zeros((16,), jnp.float32)
        return _
    lax.fori_loop(0, RCH, _zrow, None)
    for k in range((NRCH + NS - 1) // NS):
        j = s + NS * k

        @pl.when(j < NRCH)
        def _():
            pltpu.sync_copy(rowsA, acc.at[pl.ds(j * RCH, RCH)])

    plsc.subcore_barrier()

    def _gather(idx_ref, j, buf, sem):
        pltpu.async_copy(h_hbm.at[idx_ref.at[j]], buf, sem)

    def _wait(buf, sem):
        pltpu.make_async_copy(h_hbm.at[pl.ds(0, CHUNK)], buf, sem).wait()

    def _scat(idx_ref, j, buf):
        pltpu.sync_copy(buf, acc.at[idx_ref.at[j]], add=True)

    idx_pairs = [(sidxA, didxA), (sidxB, didxB)]

    pltpu.sync_copy(src_hbm.at[w, 0], sidxA)
    pltpu.sync_copy(dst_hbm.at[w, 0], didxA)
    for g in range(NGRP):
        si, di = idx_pairs[g % 2]
        sn, dn = idx_pairs[(g + 1) % 2]
        if g + 1 < NGRP:
            pltpu.async_copy(src_hbm.at[w, g + 1], sn, semI)
            pltpu.async_copy(dst_hbm.at[w, g + 1], dn, semI)

        _gather(si, 0, rowsA, semA)

        def _pair(k, _):
            _gather(si, 2 * k + 1, rowsB, semB)
            _wait(rowsA, semA)
            _scat(di, 2 * k, rowsA)
            _gather(si, 2 * k + 2, rowsA, semA)
            _wait(rowsB, semB)
            _scat(di, 2 * k + 1, rowsB)
            return _
        lax.fori_loop(0, (GRP - 1) // 2, _pair, None)
        _wait(rowsA, semA)
        _scat(di, GRP - 1, rowsA)

        if g + 1 < NGRP:
            pltpu.make_async_copy(src_hbm.at[w, 0], sn, semI).wait()
            pltpu.make_async_copy(dst_hbm.at[w, 0], dn, semI).wait()

    plsc.subcore_barrier()

    for k in range((NRCH + NS - 1) // NS):
        j = s + NS * k

        @pl.when(j < NRCH)
        def _():
            pltpu.sync_copy(acc.at[pl.ds(j * RCH, RCH)], rowsA)
            pltpu.sync_copy(rowsA, out_hbm.at[c, pl.ds(j * RCH, RCH)])


_sc_scatter = functools.partial(
    pl.kernel,
    mesh=plsc.VectorSubcoreMesh(
        core_axis_name="c", subcore_axis_name="s", num_cores=NC, num_subcores=NS
    ),
    out_type=jax.ShapeDtypeStruct((NC, N_NODES, D), jnp.float32),
    scratch_types=[
        pltpu.VMEM((GRP, CHUNK), jnp.int32),
        pltpu.VMEM((GRP, CHUNK), jnp.int32),
        pltpu.VMEM((GRP, CHUNK), jnp.int32),
        pltpu.VMEM((GRP, CHUNK), jnp.int32),
        pltpu.VMEM((CHUNK, D), jnp.float32),
        pltpu.VMEM((CHUNK, D), jnp.float32),
        pltpu.SemaphoreType.DMA,
        pltpu.SemaphoreType.DMA,
        pltpu.SemaphoreType.DMA,
        pltpu.VMEM_SHARED((ACC_ROWS, D), jnp.float32),
    ],
)(_sc_body)


RB = 2000
NB = N_NODES // RB

_HI = jax.lax.Precision.DEFAULT


def _tc_layer_body(p_ref, h_ref, z_ref, W1_ref, b1_ref, W2_ref, b2_ref,
                   pWa_ref, pWb_ref, hout_ref, zout_ref):
    h = h_ref[...]
    pooled = p_ref[0] + p_ref[1] + h
    t = jnp.dot(pooled, W1_ref[...], preferred_element_type=jnp.float32,
                precision=_HI) + b1_ref[...]
    t = jnp.maximum(t, 0.0)
    hn = jnp.dot(t, W2_ref[...], preferred_element_type=jnp.float32,
                 precision=_HI) + b2_ref[...]
    hn = jnp.maximum(hn, 0.0)
    z = z_ref[...]
    z = z + jnp.dot(h, pWb_ref[...], preferred_element_type=jnp.float32,
                    precision=_HI)
    z = z + jnp.dot(hn, pWa_ref[...], preferred_element_type=jnp.float32,
                    precision=_HI)
    hout_ref[...] = hn
    zout_ref[...] = z


def _tc_layer(p, h, z, W1f, b1f, W2f, b2f, pWa, pWb):
    full = lambda shp: pl.BlockSpec(shp, lambda i: (0,) * len(shp))
    return pl.pallas_call(
        _tc_layer_body,
        grid=(NB,),
        in_specs=[
            pl.BlockSpec((NC, RB, D), lambda i: (0, i, 0)),
            pl.BlockSpec((RB, D), lambda i: (i, 0)),
            pl.BlockSpec((RB, OUT), lambda i: (i, 0)),
            full((D, D)), full((1, D)), full((D, D)), full((1, D)),
            full((D, OUT)), full((D, OUT)),
        ],
        out_specs=[
            pl.BlockSpec((RB, D), lambda i: (i, 0)),
            pl.BlockSpec((RB, OUT), lambda i: (i, 0)),
        ],
        out_shape=[
            jax.ShapeDtypeStruct((N_NODES, D), jnp.float32),
            jax.ShapeDtypeStruct((N_NODES, OUT), jnp.float32),
        ],
    )(p, h, z, W1f, b1f, W2f, b2f, pWa, pWb)


def _tc_last_body(p_ref, h_ref, z_ref, W1_ref, b1_ref, W2_ref, b2_ref,
                  pWa_ref, b_ref, pb_ref, out_ref):
    i = pl.program_id(0)

    @pl.when(i == 0)
    def _():
        bias = jnp.sum(pb_ref[...], axis=0)
        out_ref[...] = jnp.broadcast_to(bias[None, :], (G, OUT))

    h = h_ref[...]
    pooled = p_ref[0] + p_ref[1] + h
    t = jnp.dot(pooled, W1_ref[...], preferred_element_type=jnp.float32,
                precision=_HI) + b1_ref[...]
    t = jnp.maximum(t, 0.0)
    hn = jnp.dot(t, W2_ref[...], preferred_element_type=jnp.float32,
                 precision=_HI) + b2_ref[...]
    hn = jnp.maximum(hn, 0.0)
    z = z_ref[...] + jnp.dot(hn, pWa_ref[...],
                             preferred_element_type=jnp.float32, precision=_HI)
    bvec = b_ref[0, 0, :]
    onehot = (bvec[:, None] == lax.broadcasted_iota(jnp.int32, (RB, G), 1))
    seg = lax.dot_general(onehot.astype(jnp.float32), z,
                          (((0,), (0,)), ((), ())),
                          preferred_element_type=jnp.float32, precision=_HI)
    out_ref[...] += seg


def _tc_last(p, h, z, W1f, b1f, W2f, b2f, pWa, batch3d, pred_b):
    full = lambda shp: pl.BlockSpec(shp, lambda i: (0,) * len(shp))
    return pl.pallas_call(
        _tc_last_body,
        grid=(NB,),
        in_specs=[
            pl.BlockSpec((NC, RB, D), lambda i: (0, i, 0)),
            pl.BlockSpec((RB, D), lambda i: (i, 0)),
            pl.BlockSpec((RB, OUT), lambda i: (i, 0)),
            full((D, D)), full((1, D)), full((D, D)), full((1, D)),
            full((D, OUT)),
            pl.BlockSpec((1, 1, RB), lambda i: (i, 0, 0)),
            full((L + 1, OUT)),
        ],
        out_specs=pl.BlockSpec((G, OUT), lambda i: (0, 0)),
        out_shape=jax.ShapeDtypeStruct((G, OUT), jnp.float32),
    )(p, h, z, W1f, b1f, W2f, b2f, pWa, batch3d, pred_b)


def kernel(x, edge_index, batch, mlp_W1, mlp_b1, bn1_g, bn1_b, mlp_W2, mlp_b2,
           gbn_g, gbn_b, pred_W, pred_b):
    src2d = edge_index[1].astype(jnp.int32).reshape(NW, NGRP, GRP, CHUNK)
    dst2d = edge_index[0].astype(jnp.int32).reshape(NW, NGRP, GRP, CHUNK)
    batch3d = batch.astype(jnp.int32).reshape(NB, 1, RB)

    s = 1.0 / jnp.sqrt(1.0 + BN_EPS)
    W1f = mlp_W1 * (bn1_g * s)[:, None, :]
    b1f = (mlp_b1 * bn1_g * s + bn1_b)[:, None, :]
    W2f = mlp_W2 * (gbn_g * s)[:, None, :]
    b2f = (mlp_b2 * gbn_g * s + gbn_b)[:, None, :]
    pW0_zero = jnp.zeros((D, OUT), jnp.float32)

    h = x
    z = jnp.zeros((N_NODES, OUT), jnp.float32)
    for l in range(L - 1):
        p = _sc_scatter(h, src2d, dst2d)
        pWb = pred_W[0] if l == 0 else pW0_zero
        h, z = _tc_layer(p, h, z, W1f[l], b1f[l], W2f[l], b2f[l],
                         pred_W[l + 1], pWb)
    p = _sc_scatter(h, src2d, dst2d)
    return _tc_last(p, h, z, W1f[L - 1], b1f[L - 1], W2f[L - 1], b2f[L - 1],
                    pred_W[L], batch3d, pred_b)
